# baseline (speedup 1.0000x reference)
import functools

import jax
import jax.numpy as jnp
from jax import lax
from jax.experimental import pallas as pl
from jax.experimental.pallas import tpu as pltpu
from jax.experimental.pallas import tpu_sc as plsc

D = H = W = 128
C = 16
N = 524288
V = D * H * W

NC = 2
NS = 16
NW = NC * NS
NPW = N // NW
B = 256
NBLK = NPW // B
G = B // 16
NIDX = 8 * B
ILEN = 128
NSTREAM = NIDX // ILEN

GR = V // W
RPW = GR // NW
RCH = 16
QCH = RCH * W
NCH = RPW // RCH

_mesh = plsc.VectorSubcoreMesh(core_axis_name="c", subcore_axis_name="s")


@functools.partial(
    pl.kernel,
    mesh=_mesh,
    compiler_params=pltpu.CompilerParams(use_tc_tiling_on_sc=False),
    out_type=jax.ShapeDtypeStruct((V, C), jnp.float32),
    scratch_types=[
        pltpu.VMEM((C, RCH, W), jnp.float32),
        pltpu.VMEM((QCH, C), jnp.float32),
        pltpu.SemaphoreType.DMA,
    ],
)
def _transpose_sc(feat_hbm, table_hbm, chan_v, tout_v, sem):
    wid = lax.axis_index("s") * NC + lax.axis_index("c")
    lanes = lax.iota(jnp.int32, 16)
    perms = {d: lanes ^ d for d in (1, 2, 4, 8)}
    masks = {}
    for d in (1, 2, 4, 8):
        for bit in (0, d):
            masks[(d, bit)] = (lanes & d) == bit

    def chunk_body(ch, carry):
        r0 = wid * RPW + ch * RCH
        handles = [
            pltpu.async_copy(feat_hbm.at[c, pl.ds(r0, RCH)], chan_v.at[c], sem)
            for c in range(C)
        ]
        for h in handles:
            h.wait()

        def row_body(r, c2):
            for xb in range(8):
                x0 = xb * 16
                regs = [chan_v[c, r, pl.ds(x0, 16)] for c in range(C)]
                for d in (1, 2, 4, 8):
                    regs = [
                        jnp.where(
                            masks[(d, rr & d)],
                            regs[rr],
                            jnp.take(regs[rr ^ d], perms[d]),
                        )
                        for rr in range(C)
                    ]
                lp0 = r * W + x0
                for j in range(16):
                    tout_v[lp0 + j] = regs[j]
            return c2

        lax.fori_loop(0, RCH, row_body, 0, unroll=False)
        pltpu.sync_copy(tout_v, table_hbm.at[pl.ds(r0 * W, QCH)])
        return carry

    lax.fori_loop(0, NCH, chunk_body, 0, unroll=False)


@functools.partial(
    pl.kernel,
    mesh=_mesh,
    compiler_params=pltpu.CompilerParams(use_tc_tiling_on_sc=False),
    out_type=jax.ShapeDtypeStruct((N, 1, C), jnp.float32),
    scratch_types=[
        pltpu.VMEM((B,), jnp.float32),
        pltpu.VMEM((B,), jnp.float32),
        pltpu.VMEM((B,), jnp.float32),
        pltpu.VMEM((NIDX,), jnp.int32),
        pltpu.VMEM((NIDX,), jnp.float32),
        pltpu.VMEM((NIDX, C), jnp.float32),
        pltpu.VMEM((B, 1, C), jnp.float32),
        pltpu.SemaphoreType.DMA,
    ],
)
def _sample_sc(xs_hbm, ys_hbm, zs_hbm, table_hbm, out_hbm,
               xv, yv, zv, idx_v, w_v, rows_v, out_v, sem):
    wid = lax.axis_index("s") * NC + lax.axis_index("c")
    lanes = lax.iota(jnp.int32, 16)

    def blk_body(blk, carry):
        base = wid * NPW + blk * B
        pltpu.sync_copy(xs_hbm.at[pl.ds(base, B)], xv)
        pltpu.sync_copy(ys_hbm.at[pl.ds(base, B)], yv)
        pltpu.sync_copy(zs_hbm.at[pl.ds(base, B)], zv)

        def grp_body(g, c2):
            b0 = g * 16
            cx = xv[pl.ds(b0, 16)]
            cy = yv[pl.ds(b0, 16)]
            cz = zv[pl.ds(b0, 16)]

            def axis(cu, ext):
                gg = cu * 2.0 - 1.0
                u = (gg + 1.0) * 0.5 * (ext - 1)
                u = jnp.minimum(jnp.maximum(u, 0.0), float(ext - 1))
                u0 = u.astype(jnp.int32)
                wu = u - u0.astype(jnp.float32)
                u1 = jnp.minimum(u0 + 1, ext - 1)
                return u0, u1, wu

            x0, x1, wx = axis(cx, W)
            y0, y1, wy = axis(cy, H)
            z0, z1, wz = axis(cz, D)
            wx0 = 1.0 - wx
            wy0 = 1.0 - wy
            wz0 = 1.0 - wz
            k = 0
            for dz in (0, 1):
                zi = z1 if dz else z0
                wzs = wz if dz else wz0
                for dy in (0, 1):
                    yi = y1 if dy else y0
                    wys = wy if dy else wy0
                    zy = (zi * H + yi) * W
                    wzy = wzs * wys
                    for dx in (0, 1):
                        xi = x1 if dx else x0
                        wxs = wx if dx else wx0
                        idx_v[pl.ds(k * B + b0, 16)] = zy + xi
                        w_v[pl.ds(k * B + b0, 16)] = wzy * wxs
                        k += 1
            return c2

        lax.fori_loop(0, G, grp_body, 0, unroll=False)

        handles = [
            pltpu.async_copy(
                table_hbm.at[idx_v.at[pl.ds(j * ILEN, ILEN)]],
                rows_v.at[pl.ds(j * ILEN, ILEN)],
                sem,
            )
            for j in range(NSTREAM)
        ]
        for h in handles:
            h.wait()

        def acc_body(g, c2):
            b0 = g * 16
            wks = [w_v[pl.ds(k * B + b0, 16)] for k in range(8)]
            for j in range(16):
                lane_j = jnp.full((16,), j, jnp.int32)
                acc = None
                for k in range(8):
                    row = rows_v[k * B + b0 + j]
                    wjk = jnp.take(wks[k], lane_j)
                    term = row * wjk
                    acc = term if acc is None else acc + term
                out_v[b0 + j, 0] = acc
            return c2

        lax.fori_loop(0, G, acc_body, 0, unroll=False)
        pltpu.sync_copy(out_v, out_hbm.at[pl.ds(base, B)])
        return carry

    lax.fori_loop(0, NBLK, blk_body, 0, unroll=False)


@jax.jit
def kernel(coords, features):
    feat3 = features.reshape(C, GR, W)
    table = _transpose_sc(feat3)
    xs = coords[:, 0]
    ys = coords[:, 1]
    zs = coords[:, 2]
    return _sample_sc(xs, ys, zs, table)


# speedup vs baseline: 1.7281x; 1.7281x over previous
import functools

import jax
import jax.numpy as jnp
from jax import lax
from jax.experimental import pallas as pl
from jax.experimental.pallas import tpu as pltpu
from jax.experimental.pallas import tpu_sc as plsc

D = H = W = 128
C = 16
N = 524288
V = D * H * W

NC = 2
NS = 16
NW = NC * NS
NPW = N // NW
B = 256
NBLK = NPW // B
G = B // 16
NIDX = 8 * B
ILEN = 128
NSTREAM = NIDX // ILEN

GR = V // W
RPW = GR // NW
RCH = 16
QCH = RCH * W
NCH = RPW // RCH

_mesh = plsc.VectorSubcoreMesh(core_axis_name="c", subcore_axis_name="s")


@functools.partial(
    pl.kernel,
    mesh=_mesh,
    compiler_params=pltpu.CompilerParams(use_tc_tiling_on_sc=False),
    out_type=jax.ShapeDtypeStruct((V, C), jnp.float32),
    scratch_types=[
        pltpu.VMEM((C, RCH, W), jnp.float32),
        pltpu.VMEM((QCH, C), jnp.float32),
        pltpu.SemaphoreType.DMA,
    ],
)
def _transpose_sc(feat_hbm, table_hbm, chan_v, tout_v, sem):
    wid = lax.axis_index("s") * NC + lax.axis_index("c")
    lanes = lax.iota(jnp.int32, 16)
    perms = {d: lanes ^ d for d in (1, 2, 4, 8)}
    masks = {}
    for d in (1, 2, 4, 8):
        for bit in (0, d):
            masks[(d, bit)] = (lanes & d) == bit

    def chunk_body(ch, carry):
        r0 = wid * RPW + ch * RCH
        handles = [
            pltpu.async_copy(feat_hbm.at[c, pl.ds(r0, RCH)], chan_v.at[c], sem)
            for c in range(C)
        ]
        for h in handles:
            h.wait()

        def row_body(r, c2):
            for xb in range(8):
                x0 = xb * 16
                regs = [chan_v[c, r, pl.ds(x0, 16)] for c in range(C)]
                for d in (1, 2, 4, 8):
                    regs = [
                        jnp.where(
                            masks[(d, rr & d)],
                            regs[rr],
                            jnp.take(regs[rr ^ d], perms[d]),
                        )
                        for rr in range(C)
                    ]
                lp0 = r * W + x0
                for j in range(16):
                    tout_v[lp0 + j] = regs[j]
            return c2

        lax.fori_loop(0, RCH, row_body, 0, unroll=False)
        pltpu.sync_copy(tout_v, table_hbm.at[pl.ds(r0 * W, QCH)])
        return carry

    lax.fori_loop(0, NCH, chunk_body, 0, unroll=False)


@functools.partial(
    pl.kernel,
    mesh=_mesh,
    compiler_params=pltpu.CompilerParams(use_tc_tiling_on_sc=False),
    out_type=jax.ShapeDtypeStruct((N, C), jnp.float32),
    scratch_types=[
        pltpu.VMEM((B,), jnp.float32),
        pltpu.VMEM((B,), jnp.float32),
        pltpu.VMEM((B,), jnp.float32),
        pltpu.VMEM((NIDX,), jnp.int32),
        pltpu.VMEM((NIDX,), jnp.float32),
        pltpu.VMEM((NIDX, C), jnp.float32),
        pltpu.VMEM((B, C), jnp.float32),
        pltpu.SemaphoreType.DMA,
    ],
)
def _sample_sc(xs_hbm, ys_hbm, zs_hbm, table_hbm, out_hbm,
               xv, yv, zv, idx_v, w_v, rows_v, out_v, sem):
    wid = lax.axis_index("s") * NC + lax.axis_index("c")
    lanes = lax.iota(jnp.int32, 16)

    def blk_body(blk, carry):
        base = wid * NPW + blk * B
        pltpu.sync_copy(xs_hbm.at[pl.ds(base, B)], xv)
        pltpu.sync_copy(ys_hbm.at[pl.ds(base, B)], yv)
        pltpu.sync_copy(zs_hbm.at[pl.ds(base, B)], zv)

        def grp_body(g, c2):
            b0 = g * 16
            cx = xv[pl.ds(b0, 16)]
            cy = yv[pl.ds(b0, 16)]
            cz = zv[pl.ds(b0, 16)]

            def axis(cu, ext):
                gg = cu * 2.0 - 1.0
                u = (gg + 1.0) * 0.5 * (ext - 1)
                u = jnp.minimum(jnp.maximum(u, 0.0), float(ext - 1))
                u0 = u.astype(jnp.int32)
                wu = u - u0.astype(jnp.float32)
                u1 = jnp.minimum(u0 + 1, ext - 1)
                return u0, u1, wu

            x0, x1, wx = axis(cx, W)
            y0, y1, wy = axis(cy, H)
            z0, z1, wz = axis(cz, D)
            wx0 = 1.0 - wx
            wy0 = 1.0 - wy
            wz0 = 1.0 - wz
            k = 0
            for dz in (0, 1):
                zi = z1 if dz else z0
                wzs = wz if dz else wz0
                for dy in (0, 1):
                    yi = y1 if dy else y0
                    wys = wy if dy else wy0
                    zy = (zi * H + yi) * W
                    wzy = wzs * wys
                    for dx in (0, 1):
                        xi = x1 if dx else x0
                        wxs = wx if dx else wx0
                        idx_v[pl.ds(k * B + b0, 16)] = zy + xi
                        w_v[pl.ds(k * B + b0, 16)] = wzy * wxs
                        k += 1
            return c2

        lax.fori_loop(0, G, grp_body, 0, unroll=False)

        handles = [
            pltpu.async_copy(
                table_hbm.at[idx_v.at[pl.ds(j * ILEN, ILEN)]],
                rows_v.at[pl.ds(j * ILEN, ILEN)],
                sem,
            )
            for j in range(NSTREAM)
        ]
        for h in handles:
            h.wait()

        def acc_body(g, c2):
            b0 = g * 16
            wks = [w_v[pl.ds(k * B + b0, 16)] for k in range(8)]
            for j in range(16):
                lane_j = jnp.full((16,), j, jnp.int32)
                acc = None
                for k in range(8):
                    row = rows_v[k * B + b0 + j]
                    wjk = jnp.take(wks[k], lane_j)
                    term = row * wjk
                    acc = term if acc is None else acc + term
                out_v[b0 + j] = acc
            return c2

        lax.fori_loop(0, G, acc_body, 0, unroll=False)
        pltpu.sync_copy(out_v, out_hbm.at[pl.ds(base, B)])
        return carry

    lax.fori_loop(0, NBLK, blk_body, 0, unroll=False)


@jax.jit
def kernel(coords, features):
    feat3 = features.reshape(C, GR, W)
    table = _transpose_sc(feat3)
    xs = coords[:, 0]
    ys = coords[:, 1]
    zs = coords[:, 2]
    out = _sample_sc(xs, ys, zs, table)
    return out[:, None, :]


# speedup vs baseline: 2.2892x; 1.3247x over previous
import functools

import jax
import jax.numpy as jnp
from jax import lax
from jax.experimental import pallas as pl
from jax.experimental.pallas import tpu as pltpu
from jax.experimental.pallas import tpu_sc as plsc

D = H = W = 128
C = 16
N = 524288
V = D * H * W

NC = 2
NS = 16
NW = NC * NS
NPW = N // NW
B = 256
NBLK = NPW // B
G = B // 16
NIDX = 8 * B
ILEN = 128
NSTREAM = NIDX // ILEN

GR = V // W
RPW = GR // NW
RCH = 16
QCH = RCH * W
NCH = RPW // RCH

_mesh = plsc.VectorSubcoreMesh(core_axis_name="c", subcore_axis_name="s")


@functools.partial(
    pl.kernel,
    mesh=_mesh,
    compiler_params=pltpu.CompilerParams(use_tc_tiling_on_sc=False),
    out_type=jax.ShapeDtypeStruct((V, C), jnp.float32),
    scratch_types=[
        pltpu.VMEM((C, RCH, W), jnp.float32),
        pltpu.VMEM((QCH, C), jnp.float32),
        pltpu.SemaphoreType.DMA,
    ],
)
def _transpose_sc(feat_hbm, table_hbm, chan_v, tout_v, sem):
    wid = lax.axis_index("s") * NC + lax.axis_index("c")
    lanes = lax.iota(jnp.int32, 16)
    perms = {d: lanes ^ d for d in (1, 2, 4, 8)}
    masks = {}
    for d in (1, 2, 4, 8):
        for bit in (0, d):
            masks[(d, bit)] = (lanes & d) == bit

    def chunk_body(ch, carry):
        r0 = wid * RPW + ch * RCH
        handles = [
            pltpu.async_copy(feat_hbm.at[c, pl.ds(r0, RCH)], chan_v.at[c], sem)
            for c in range(C)
        ]
        for h in handles:
            h.wait()

        def row_body(r, c2):
            for xb in range(8):
                x0 = xb * 16
                regs = [chan_v[c, r, pl.ds(x0, 16)] for c in range(C)]
                for d in (1, 2, 4, 8):
                    regs = [
                        jnp.where(
                            masks[(d, rr & d)],
                            regs[rr],
                            jnp.take(regs[rr ^ d], perms[d]),
                        )
                        for rr in range(C)
                    ]
                lp0 = r * W + x0
                for j in range(16):
                    tout_v[lp0 + j] = regs[j]
            return c2

        lax.fori_loop(0, RCH, row_body, 0, unroll=False)
        pltpu.sync_copy(tout_v, table_hbm.at[pl.ds(r0 * W, QCH)])
        return carry

    lax.fori_loop(0, NCH, chunk_body, 0, unroll=False)


@functools.partial(
    pl.kernel,
    mesh=_mesh,
    compiler_params=pltpu.CompilerParams(use_tc_tiling_on_sc=False),
    out_type=jax.ShapeDtypeStruct((N, C), jnp.float32),
    scratch_types=[
        pltpu.VMEM((B,), jnp.float32),
        pltpu.VMEM((B,), jnp.float32),
        pltpu.VMEM((B,), jnp.float32),
        pltpu.VMEM((B,), jnp.float32),
        pltpu.VMEM((B,), jnp.float32),
        pltpu.VMEM((B,), jnp.float32),
        pltpu.VMEM((NIDX,), jnp.int32),
        pltpu.VMEM((NIDX,), jnp.int32),
        pltpu.VMEM((NIDX,), jnp.float32),
        pltpu.VMEM((NIDX,), jnp.float32),
        pltpu.VMEM((NIDX, C), jnp.float32),
        pltpu.VMEM((NIDX, C), jnp.float32),
        pltpu.VMEM((B, C), jnp.float32),
        pltpu.VMEM((B, C), jnp.float32),
        pltpu.SemaphoreType.DMA,
        pltpu.SemaphoreType.DMA,
        pltpu.SemaphoreType.DMA,
        pltpu.SemaphoreType.DMA,
        pltpu.SemaphoreType.DMA,
        pltpu.SemaphoreType.DMA,
    ],
)
def _sample_sc(xs_hbm, ys_hbm, zs_hbm, table_hbm, out_hbm,
               xv0, yv0, zv0, xv1, yv1, zv1, idx0, idx1, w0, w1,
               rows0, rows1, out0, out1, sem0, sem1,
               csem0, csem1, osem0, osem1):
    wid = lax.axis_index("s") * NC + lax.axis_index("c")
    lanes = lax.iota(jnp.int32, 16)
    cv = [(xv0, yv0, zv0), (xv1, yv1, zv1)]
    idxb = [idx0, idx1]
    wb = [w0, w1]
    rowsb = [rows0, rows1]
    outb = [out0, out1]
    semb = [sem0, sem1]
    csemb = [csem0, csem1]
    osemb = [osem0, osem1]

    def fire_coords(blk, buf):
        bc = jnp.minimum(blk, NBLK - 1)
        base = wid * NPW + bc * B
        pltpu.async_copy(xs_hbm.at[pl.ds(base, B)], cv[buf][0], csemb[buf])
        pltpu.async_copy(ys_hbm.at[pl.ds(base, B)], cv[buf][1], csemb[buf])
        pltpu.async_copy(zs_hbm.at[pl.ds(base, B)], cv[buf][2], csemb[buf])

    def wait_coords(buf):
        for r in cv[buf]:
            pltpu.make_async_copy(xs_hbm.at[pl.ds(0, B)], r, csemb[buf]).wait()

    def idx_weights(buf):
        xv, yv, zv = cv[buf]
        idx_v = idxb[buf]
        w_v = wb[buf]

        def grp_body(g, c2):
            b0 = g * 16
            cx = xv[pl.ds(b0, 16)]
            cy = yv[pl.ds(b0, 16)]
            cz = zv[pl.ds(b0, 16)]

            def axis(cu, ext):
                gg = cu * 2.0 - 1.0
                u = (gg + 1.0) * 0.5 * (ext - 1)
                u = jnp.minimum(jnp.maximum(u, 0.0), float(ext - 1))
                u0 = u.astype(jnp.int32)
                wu = u - u0.astype(jnp.float32)
                u1 = jnp.minimum(u0 + 1, ext - 1)
                return u0, u1, wu

            x0, x1, wx = axis(cx, W)
            y0, y1, wy = axis(cy, H)
            z0, z1, wz = axis(cz, D)
            wx0 = 1.0 - wx
            wy0 = 1.0 - wy
            wz0 = 1.0 - wz
            k = 0
            for dz in (0, 1):
                zi = z1 if dz else z0
                wzs = wz if dz else wz0
                for dy in (0, 1):
                    yi = y1 if dy else y0
                    wys = wy if dy else wy0
                    zy = (zi * H + yi) * W
                    wzy = wzs * wys
                    for dx in (0, 1):
                        xi = x1 if dx else x0
                        wxs = wx if dx else wx0
                        idx_v[pl.ds(k * B + b0, 16)] = zy + xi
                        w_v[pl.ds(k * B + b0, 16)] = wzy * wxs
                        k += 1
            return c2

        lax.fori_loop(0, G, grp_body, 0, unroll=False)

    def fire_gathers(buf):
        for j in range(NSTREAM):
            pltpu.async_copy(
                table_hbm.at[idxb[buf].at[pl.ds(j * ILEN, ILEN)]],
                rowsb[buf].at[pl.ds(j * ILEN, ILEN)],
                semb[buf],
            )

    def wait_gathers(buf):
        for j in range(NSTREAM):
            pltpu.make_async_copy(
                table_hbm.at[idxb[buf].at[pl.ds(j * ILEN, ILEN)]],
                rowsb[buf].at[pl.ds(j * ILEN, ILEN)],
                semb[buf],
            ).wait()

    def accum(buf):
        w_v = wb[buf]
        rows_v = rowsb[buf]
        out_v = outb[buf]

        def acc_body(g, c2):
            b0 = g * 16
            wks = [w_v[pl.ds(k * B + b0, 16)] for k in range(8)]
            for j in range(16):
                lane_j = jnp.full((16,), j, jnp.int32)
                acc = None
                for k in range(8):
                    row = rows_v[k * B + b0 + j]
                    wjk = jnp.take(wks[k], lane_j)
                    term = row * wjk
                    acc = term if acc is None else acc + term
                out_v[b0 + j] = acc
            return c2

        lax.fori_loop(0, G, acc_body, 0, unroll=False)

    def fire_out(blk, buf):
        base = wid * NPW + blk * B
        pltpu.async_copy(outb[buf], out_hbm.at[pl.ds(base, B)], osemb[buf])

    def wait_out(buf):
        pltpu.make_async_copy(outb[buf], out_hbm.at[pl.ds(0, B)],
                              osemb[buf]).wait()

    def sub_block(i, buf):
        nbuf = 1 - buf
        wait_coords(nbuf)
        idx_weights(nbuf)
        fire_gathers(nbuf)
        fire_coords(i + 2, buf)
        wait_gathers(buf)

        @pl.when(i >= 2)
        def _():
            wait_out(buf)

        accum(buf)
        fire_out(i, buf)

    fire_coords(jnp.int32(0), 0)
    fire_coords(jnp.int32(1), 1)
    wait_coords(0)
    idx_weights(0)
    fire_gathers(0)

    def body(d, carry):
        sub_block(2 * d, 0)
        sub_block(2 * d + 1, 1)
        return carry

    lax.fori_loop(0, NBLK // 2, body, 0, unroll=False)

    wait_gathers(0)
    wait_coords(1)
    wait_out(0)
    wait_out(1)


@jax.jit
def kernel(coords, features):
    feat3 = features.reshape(C, GR, W)
    table = _transpose_sc(feat3)
    xs = coords[:, 0]
    ys = coords[:, 1]
    zs = coords[:, 2]
    out = _sample_sc(xs, ys, zs, table)
    return out[:, None, :]
